# baseline (speedup 1.0000x reference)
import functools

import jax
import jax.numpy as jnp
from jax.experimental import pallas as pl
from jax.experimental.pallas import tpu as pltpu


def _se_row_kernel(x_ref, w1_ref, b1_ref, w2_ref, b2_ref, o_ref, *, inv_hw):
    x = x_ref[...]
    hw = x.shape[1]
    ones = jnp.full((hw, 1), inv_hw, jnp.float32)
    s = jnp.dot(x, ones, preferred_element_type=jnp.float32)
    z1 = jnp.dot(w1_ref[...], s,
                 preferred_element_type=jnp.float32) + b1_ref[...]
    z1 = jnp.maximum(z1, 0.0)
    z2 = jnp.dot(w2_ref[...], z1,
                 preferred_element_type=jnp.float32) + b2_ref[...]
    gate = jax.nn.sigmoid(z2)
    o_ref[...] = x * gate


def kernel(x_nchw, w1, b1, w2, b2):
    n, c, h, w = x_nchw.shape
    hw = h * w
    mid = w1.shape[0]
    x2 = x_nchw.reshape(n * c, hw)
    b1c = b1.reshape(mid, 1)
    b2c = b2.reshape(c, 1)

    out = pl.pallas_call(
        functools.partial(_se_row_kernel, inv_hw=1.0 / hw),
        grid=(n,),
        in_specs=[
            pl.BlockSpec((c, hw), lambda i: (i, 0)),
            pl.BlockSpec((mid, c), lambda i: (0, 0)),
            pl.BlockSpec((mid, 1), lambda i: (0, 0)),
            pl.BlockSpec((c, mid), lambda i: (0, 0)),
            pl.BlockSpec((c, 1), lambda i: (0, 0)),
        ],
        out_specs=pl.BlockSpec((c, hw), lambda i: (i, 0)),
        out_shape=jax.ShapeDtypeStruct((n * c, hw), x_nchw.dtype),
        compiler_params=pltpu.CompilerParams(
            dimension_semantics=("parallel",),
            vmem_limit_bytes=48 * 1024 * 1024),
        cost_estimate=pl.CostEstimate(
            flops=int(2 * n * c * hw + 2 * n * (c * mid + mid * c)),
            transcendentals=int(n * c),
            bytes_accessed=int(4 * 2 * n * c * hw)),
    )(x2, w1, b1c, w2, b2c)
    return out.reshape(n, c, h, w)


# speedup vs baseline: 2.3909x; 2.3909x over previous
import functools

import jax
import jax.numpy as jnp
from jax.experimental import pallas as pl
from jax.experimental.pallas import tpu as pltpu

_ROWS_PER_STEP = 4


def _se_kernel(x_ref, w1_ref, b1_ref, w2_ref, b2_ref, o_ref, *, inv_hw):
    x = x_ref[...]
    s = jnp.sum(x, axis=2) * inv_hw
    z1 = jax.lax.dot_general(s, w1_ref[...], (((1,), (1,)), ((), ())),
                             preferred_element_type=jnp.float32)
    z1 = jnp.maximum(z1 + b1_ref[...], 0.0)
    z2 = jax.lax.dot_general(z1, w2_ref[...], (((1,), (1,)), ((), ())),
                             preferred_element_type=jnp.float32)
    gate = jax.nn.sigmoid(z2 + b2_ref[...])
    o_ref[...] = x * gate[:, :, None]


def kernel(x_nchw, w1, b1, w2, b2):
    n, c, h, w = x_nchw.shape
    hw = h * w
    mid = w1.shape[0]
    x3 = x_nchw.reshape(n, c, hw)
    b1r = b1.reshape(1, mid)
    b2r = b2.reshape(1, c)

    b = _ROWS_PER_STEP
    while n % b:
        b //= 2

    out = pl.pallas_call(
        functools.partial(_se_kernel, inv_hw=1.0 / hw),
        grid=(n // b,),
        in_specs=[
            pl.BlockSpec((b, c, hw), lambda i: (i, 0, 0)),
            pl.BlockSpec((mid, c), lambda i: (0, 0)),
            pl.BlockSpec((1, mid), lambda i: (0, 0)),
            pl.BlockSpec((c, mid), lambda i: (0, 0)),
            pl.BlockSpec((1, c), lambda i: (0, 0)),
        ],
        out_specs=pl.BlockSpec((b, c, hw), lambda i: (i, 0, 0)),
        out_shape=jax.ShapeDtypeStruct((n, c, hw), x_nchw.dtype),
        compiler_params=pltpu.CompilerParams(
            dimension_semantics=("arbitrary",),
            vmem_limit_bytes=60 * 1024 * 1024),
        cost_estimate=pl.CostEstimate(
            flops=int(2 * n * c * hw + 2 * n * (c * mid + mid * c)),
            transcendentals=int(n * c),
            bytes_accessed=int(4 * 2 * n * c * hw)),
    )(x3, w1, b1r, w2, b2r)
    return out.reshape(n, c, h, w)
